# Initial kernel scaffold; baseline (speedup 1.0000x reference)
import jax
import jax.numpy as jnp
from jax.experimental import pallas as pl


def kernel(x):
    raise NotImplementedError("write your pallas kernel here")


# speedup vs baseline: 18.1027x; 18.1027x over previous
import jax
import jax.numpy as jnp
from jax import lax
from jax.experimental import pallas as pl

_BLOCK_N = 256


def _scale_body(x_ref, o_ref):
    xb = x_ref[...]
    i = lax.broadcasted_iota(jnp.int32, xb.shape, 1)
    scale = jnp.where(
        i == 0, 2.0, jnp.where(i == 1, 3.0, jnp.where(i == 2, 4.0, 1.0))
    )
    o_ref[...] = xb * scale


def kernel(x):
    n, m, d = x.shape
    grid = (n // _BLOCK_N,)
    return pl.pallas_call(
        _scale_body,
        grid=grid,
        in_specs=[pl.BlockSpec((_BLOCK_N, m, d), lambda i: (i, 0, 0))],
        out_specs=pl.BlockSpec((_BLOCK_N, m, d), lambda i: (i, 0, 0)),
        out_shape=jax.ShapeDtypeStruct(x.shape, x.dtype),
    )(x)


# speedup vs baseline: 18.3338x; 1.0128x over previous
import jax
import jax.numpy as jnp
from jax import lax
from jax.experimental import pallas as pl

_BLOCK_N = 512


def _scale_body(x_ref, o_ref):
    xb = x_ref[...]
    i = lax.broadcasted_iota(jnp.int32, xb.shape, 1)
    scale = jnp.where(
        i == 0, 2.0, jnp.where(i == 1, 3.0, jnp.where(i == 2, 4.0, 1.0))
    )
    o_ref[...] = xb * scale


def kernel(x):
    n, m, d = x.shape
    grid = (n // _BLOCK_N,)
    return pl.pallas_call(
        _scale_body,
        grid=grid,
        in_specs=[pl.BlockSpec((_BLOCK_N, m, d), lambda i: (i, 0, 0))],
        out_specs=pl.BlockSpec((_BLOCK_N, m, d), lambda i: (i, 0, 0)),
        out_shape=jax.ShapeDtypeStruct(x.shape, x.dtype),
    )(x)
